# baseline (speedup 1.0000x reference)
import functools

import jax
import jax.numpy as jnp
import numpy as np
from jax.experimental import pallas as pl
from jax.experimental.pallas import tpu as pltpu

_PMIN = np.float32(0.1)
_PMAX = np.float32(1.0)
_EPS_D = np.float32(1e-12)
_TINY = np.float32(np.finfo(np.float32).tiny)


def _stats_kernel(x_ref, out_ref, *, n_cols, block_w):
    c = pl.program_id(0)
    x = x_ref[...]
    a = jnp.abs(x)
    col = jax.lax.broadcasted_iota(jnp.int32, x.shape, 1) + c * block_w
    valid = col < n_cols
    inf = jnp.float32(jnp.inf)
    mn = jnp.min(jnp.where(valid, a, inf), axis=-1, keepdims=True)
    mx = jnp.max(jnp.where(valid, a, -inf), axis=-1, keepdims=True)
    s1 = jnp.sum(jnp.where(valid, a, 0.0), axis=-1, keepdims=True)
    s2 = jnp.sum(jnp.where(valid, a * a, 0.0), axis=-1, keepdims=True)

    @pl.when(c == 0)
    def _():
        out_ref[:, 0:1] = mn
        out_ref[:, 1:2] = mx
        out_ref[:, 2:3] = s1
        out_ref[:, 3:4] = s2

    @pl.when(c != 0)
    def _():
        out_ref[:, 0:1] = jnp.minimum(out_ref[:, 0:1], mn)
        out_ref[:, 1:2] = jnp.maximum(out_ref[:, 1:2], mx)
        out_ref[:, 2:3] = out_ref[:, 2:3] + s1
        out_ref[:, 3:4] = out_ref[:, 3:4] + s2


def _threefry_xor(x1):
    ks = (jnp.uint32(0), jnp.uint32(1), jnp.uint32(0x1BD11BDB))
    rot = ((13, 15, 26, 6), (17, 29, 16, 24))
    x0 = jnp.zeros_like(x1)
    x1 = x1 + ks[1]
    for i in range(5):
        for r in rot[i % 2]:
            x0 = x0 + x1
            x1 = (x1 << jnp.uint32(r)) | (x1 >> jnp.uint32(32 - r))
            x1 = x1 ^ x0
        x0 = x0 + ks[(i + 1) % 3]
        x1 = x1 + ks[(i + 2) % 3] + jnp.uint32(i + 1)
    return x0 ^ x1


def _main_kernel(x_ref, stats_ref, probs_ref, idx_ref, bz_ref, bi_ref, *,
                 n_cols, block_w):
    c = pl.program_id(0)
    nc = pl.num_programs(0)
    x = x_ref[...]
    a = jnp.abs(x)
    gmin = stats_ref[:, 0:1]
    gmax = stats_ref[:, 1:2]
    s1 = stats_ref[:, 2:3]
    s2 = stats_ref[:, 3:4]
    denom = gmax - gmin + _EPS_D
    nf = jnp.float32(n_cols)
    sumq = (s2 - 2.0 * gmin * s1 + nf * gmin * gmin) / (denom * denom)
    psum = (_PMAX - _PMIN) * sumq + _PMIN * nf

    t = (a - gmin) / denom
    p = t * t * (_PMAX - _PMIN) + _PMIN
    probs_ref[...] = p / psum

    coli = jax.lax.broadcasted_iota(jnp.int32, x.shape, 1) + c * block_w
    row = jax.lax.broadcasted_iota(jnp.uint32, x.shape, 0)
    f = row * jnp.uint32(n_cols) + coli.astype(jnp.uint32)
    bits = _threefry_xor(f)
    fb = (bits >> jnp.uint32(9)) | jnp.uint32(0x3F800000)
    fl = jax.lax.bitcast_convert_type(fb, jnp.float32) - jnp.float32(1.0)
    u = jnp.maximum(_TINY, fl * (jnp.float32(1.0) - _TINY) + _TINY)
    w = -jnp.log(u)
    z = p / w
    z = jnp.where(coli < n_cols, z, jnp.float32(-1.0))
    zmax = jnp.max(z, axis=-1, keepdims=True)
    li = jnp.min(jnp.where(z == zmax, coli, jnp.int32(n_cols)),
                 axis=-1, keepdims=True)

    @pl.when(c == 0)
    def _():
        bz_ref[...] = zmax
        bi_ref[...] = li

    @pl.when(c != 0)
    def _():
        upd = zmax > bz_ref[...]
        bz_ref[...] = jnp.where(upd, zmax, bz_ref[...])
        bi_ref[...] = jnp.where(upd, li, bi_ref[...])

    @pl.when(c == nc - 1)
    def _():
        idx_ref[...] = bi_ref[...]


def kernel(igrad, *, block_w=8192, interpret=False):
    B, N = igrad.shape
    C = pl.cdiv(N, block_w)
    stats = pl.pallas_call(
        functools.partial(_stats_kernel, n_cols=N, block_w=block_w),
        grid=(C,),
        in_specs=[pl.BlockSpec((B, block_w), lambda c: (0, c))],
        out_specs=pl.BlockSpec((B, 4), lambda c: (0, 0)),
        out_shape=jax.ShapeDtypeStruct((B, 4), jnp.float32),
        interpret=interpret,
    )(igrad)
    probs, idx2 = pl.pallas_call(
        functools.partial(_main_kernel, n_cols=N, block_w=block_w),
        grid=(C,),
        in_specs=[pl.BlockSpec((B, block_w), lambda c: (0, c)),
                  pl.BlockSpec((B, 4), lambda c: (0, 0))],
        out_specs=[pl.BlockSpec((B, block_w), lambda c: (0, c)),
                   pl.BlockSpec((B, 1), lambda c: (0, 0))],
        out_shape=[jax.ShapeDtypeStruct((B, N), jnp.float32),
                   jax.ShapeDtypeStruct((B, 1), jnp.int32)],
        scratch_shapes=[pltpu.VMEM((B, 1), jnp.float32),
                        pltpu.VMEM((B, 1), jnp.int32)],
        interpret=interpret,
    )(igrad, stats)
    return probs, idx2.reshape(B)


# speedup vs baseline: 1.3986x; 1.3986x over previous
import functools

import jax
import jax.numpy as jnp
import numpy as np
from jax.experimental import pallas as pl
from jax.experimental.pallas import tpu as pltpu

_PMIN = np.float32(0.1)
_PMAX = np.float32(1.0)
_EPS_D = np.float32(1e-12)
_TINY = np.float32(np.finfo(np.float32).tiny)
_LANES = 128


def _stats_kernel(x_ref, out_ref, mn_ref, mx_ref, s1_ref, s2_ref, *,
                  n_cols, block_w):
    c = pl.program_id(0)
    nc = pl.num_programs(0)
    B = x_ref.shape[0]
    nfold = block_w // _LANES

    accmn = jnp.where(c == 0, jnp.full((B, _LANES), jnp.inf, jnp.float32),
                      mn_ref[...])
    accmx = jnp.where(c == 0, jnp.zeros((B, _LANES), jnp.float32),
                      mx_ref[...])
    accs1 = jnp.where(c == 0, jnp.zeros((B, _LANES), jnp.float32),
                      s1_ref[...])
    accs2 = jnp.where(c == 0, jnp.zeros((B, _LANES), jnp.float32),
                      s2_ref[...])

    base = c * block_w
    lane = jax.lax.broadcasted_iota(jnp.int32, (B, _LANES), 1)
    for k in range(nfold):
        a = jnp.abs(x_ref[:, k * _LANES:(k + 1) * _LANES])
        valid = (base + k * _LANES) + lane < n_cols
        am = jnp.where(valid, a, 0.0)
        accmn = jnp.minimum(accmn, jnp.where(valid, a, jnp.inf))
        accmx = jnp.maximum(accmx, am)
        accs1 = accs1 + am
        accs2 = accs2 + am * am

    mn_ref[...] = accmn
    mx_ref[...] = accmx
    s1_ref[...] = accs1
    s2_ref[...] = accs2

    @pl.when(c == nc - 1)
    def _():
        out_ref[:, 0:1] = jnp.min(accmn, axis=-1, keepdims=True)
        out_ref[:, 1:2] = jnp.max(accmx, axis=-1, keepdims=True)
        out_ref[:, 2:3] = jnp.sum(accs1, axis=-1, keepdims=True)
        out_ref[:, 3:4] = jnp.sum(accs2, axis=-1, keepdims=True)


def _threefry_xor(x1):
    ks = (jnp.uint32(0), jnp.uint32(1), jnp.uint32(0x1BD11BDB))
    rot = ((13, 15, 26, 6), (17, 29, 16, 24))
    x0 = jnp.zeros_like(x1)
    x1 = x1 + ks[1]
    for i in range(5):
        for r in rot[i % 2]:
            x0 = x0 + x1
            x1 = (x1 << jnp.uint32(r)) | (x1 >> jnp.uint32(32 - r))
            x1 = x1 ^ x0
        x0 = x0 + ks[(i + 1) % 3]
        x1 = x1 + ks[(i + 2) % 3] + jnp.uint32(i + 1)
    return x0 ^ x1


def _main_kernel(x_ref, stats_ref, probs_ref, idx_ref, accz_ref, acci_ref, *,
                 n_cols, block_w, sub_w):
    c = pl.program_id(0)
    nc = pl.num_programs(0)
    B = x_ref.shape[0]

    gmin = stats_ref[:, 0:1]
    gmax = stats_ref[:, 1:2]
    s1 = stats_ref[:, 2:3]
    s2 = stats_ref[:, 3:4]
    denom = gmax - gmin + _EPS_D
    nf = jnp.float32(n_cols)
    sumq = (s2 - 2.0 * gmin * s1 + nf * gmin * gmin) / (denom * denom)
    psum = (_PMAX - _PMIN) * sumq + _PMIN * nf
    inv_psum = 1.0 / psum

    accz = jnp.where(c == 0, jnp.full((B, _LANES), -1.0, jnp.float32),
                     accz_ref[...])
    acci = jnp.where(c == 0, jnp.zeros((B, _LANES), jnp.int32),
                     acci_ref[...])

    base = c * block_w
    rowc = jax.lax.broadcasted_iota(jnp.uint32, (B, sub_w), 0) \
        * jnp.uint32(n_cols)

    for s in range(block_w // sub_w):
        off = s * sub_w
        xs = x_ref[:, off:off + sub_w]
        a = jnp.abs(xs)
        t = (a - gmin) / denom
        p = t * t * (_PMAX - _PMIN) + _PMIN
        probs_ref[:, off:off + sub_w] = p * inv_psum

        coli = jax.lax.broadcasted_iota(jnp.int32, (B, sub_w), 1) \
            + (base + off)
        f = rowc + coli.astype(jnp.uint32)
        bits = _threefry_xor(f)
        fb = (bits >> jnp.uint32(9)) | jnp.uint32(0x3F800000)
        fl = jax.lax.bitcast_convert_type(fb, jnp.float32) - jnp.float32(1.0)
        u = jnp.maximum(_TINY, fl * (jnp.float32(1.0) - _TINY) + _TINY)
        w = -jnp.log(u)
        z = p / w

        for k in range(sub_w // _LANES):
            zk = z[:, k * _LANES:(k + 1) * _LANES]
            colk = coli[:, k * _LANES:(k + 1) * _LANES]
            upd = (zk > accz) & (colk < n_cols)
            accz = jnp.where(upd, zk, accz)
            acci = jnp.where(upd, colk, acci)

    accz_ref[...] = accz
    acci_ref[...] = acci

    @pl.when(c == nc - 1)
    def _():
        zmax = jnp.max(accz, axis=-1, keepdims=True)
        li = jnp.min(jnp.where(accz == zmax, acci, jnp.int32(n_cols)),
                     axis=-1, keepdims=True)
        idx_ref[...] = li


def kernel(igrad):
    B, N = igrad.shape
    w1 = 8192
    c1 = pl.cdiv(N, w1)
    stats = pl.pallas_call(
        functools.partial(_stats_kernel, n_cols=N, block_w=w1),
        grid=(c1,),
        in_specs=[pl.BlockSpec((B, w1), lambda c: (0, c))],
        out_specs=pl.BlockSpec((B, 4), lambda c: (0, 0)),
        out_shape=jax.ShapeDtypeStruct((B, 4), jnp.float32),
        scratch_shapes=[pltpu.VMEM((B, _LANES), jnp.float32)
                        for _ in range(4)],
    )(igrad)

    w2 = 2048
    sub_w = 256
    c2 = pl.cdiv(N, w2)
    probs, idx2 = pl.pallas_call(
        functools.partial(_main_kernel, n_cols=N, block_w=w2, sub_w=sub_w),
        grid=(c2,),
        in_specs=[pl.BlockSpec((B, w2), lambda c: (0, c)),
                  pl.BlockSpec((B, 4), lambda c: (0, 0))],
        out_specs=[pl.BlockSpec((B, w2), lambda c: (0, c)),
                   pl.BlockSpec((B, 1), lambda c: (0, 0))],
        out_shape=[jax.ShapeDtypeStruct((B, N), jnp.float32),
                   jax.ShapeDtypeStruct((B, 1), jnp.int32)],
        scratch_shapes=[pltpu.VMEM((B, _LANES), jnp.float32),
                        pltpu.VMEM((B, _LANES), jnp.int32)],
    )(igrad, stats)
    return probs, idx2.reshape(B)


# speedup vs baseline: 1.4324x; 1.0242x over previous
import functools

import jax
import jax.numpy as jnp
import numpy as np
from jax.experimental import pallas as pl
from jax.experimental.pallas import tpu as pltpu

_PMIN = np.float32(0.1)
_PMAX = np.float32(1.0)
_EPS_D = np.float32(1e-12)
_TINY = np.float32(np.finfo(np.float32).tiny)
_LANES = 128


def _stats_kernel(x_ref, out_ref, mn_ref, mx_ref, s1_ref, s2_ref, *,
                  n_cols, block_w):
    c = pl.program_id(0)
    nc = pl.num_programs(0)
    B = x_ref.shape[0]
    nfold = block_w // _LANES

    accmn = jnp.where(c == 0, jnp.full((B, _LANES), jnp.inf, jnp.float32),
                      mn_ref[...])
    accmx = jnp.where(c == 0, jnp.zeros((B, _LANES), jnp.float32),
                      mx_ref[...])
    accs1 = jnp.where(c == 0, jnp.zeros((B, _LANES), jnp.float32),
                      s1_ref[...])
    accs2 = jnp.where(c == 0, jnp.zeros((B, _LANES), jnp.float32),
                      s2_ref[...])

    base = c * block_w
    lane = jax.lax.broadcasted_iota(jnp.int32, (B, _LANES), 1)
    for k in range(nfold):
        a = jnp.abs(x_ref[:, k * _LANES:(k + 1) * _LANES])
        valid = (base + k * _LANES) + lane < n_cols
        am = jnp.where(valid, a, 0.0)
        accmn = jnp.minimum(accmn, jnp.where(valid, a, jnp.inf))
        accmx = jnp.maximum(accmx, am)
        accs1 = accs1 + am
        accs2 = accs2 + am * am

    mn_ref[...] = accmn
    mx_ref[...] = accmx
    s1_ref[...] = accs1
    s2_ref[...] = accs2

    @pl.when(c == nc - 1)
    def _():
        gmin = jnp.min(accmn, axis=-1, keepdims=True)
        gmax = jnp.max(accmx, axis=-1, keepdims=True)
        s1 = jnp.sum(accs1, axis=-1, keepdims=True)
        s2 = jnp.sum(accs2, axis=-1, keepdims=True)
        denom = gmax - gmin + _EPS_D
        inv_denom = 1.0 / denom
        nf = jnp.float32(n_cols)
        sumq = (s2 - 2.0 * gmin * s1 + nf * gmin * gmin) \
            * (inv_denom * inv_denom)
        psum = (_PMAX - _PMIN) * sumq + _PMIN * nf
        out_ref[:, 0:1] = -gmin * inv_denom
        out_ref[:, 1:2] = inv_denom
        out_ref[:, 2:3] = 1.0 / psum
        out_ref[:, 3:4] = psum


def _threefry_xor(x1):
    ks = (jnp.uint32(0), jnp.uint32(1), jnp.uint32(0x1BD11BDB))
    rot = ((13, 15, 26, 6), (17, 29, 16, 24))
    x0 = jnp.zeros_like(x1)
    x1 = x1 + ks[1]
    for i in range(5):
        for r in rot[i % 2]:
            x0 = x0 + x1
            x1 = (x1 << jnp.uint32(r)) | (x1 >> jnp.uint32(32 - r))
            x1 = x1 ^ x0
        x0 = x0 + ks[(i + 1) % 3]
        x1 = x1 + ks[(i + 2) % 3] + jnp.uint32(i + 1)
    return x0 ^ x1


def _main_kernel(x_ref, stats_ref, probs_ref, idx_ref, accz_ref, acci_ref, *,
                 n_cols, block_w, sub_w):
    c = pl.program_id(0)
    nc = pl.num_programs(0)
    B = x_ref.shape[0]

    c0 = stats_ref[:, 0:1]
    c1 = stats_ref[:, 1:2]
    inv_psum = stats_ref[:, 2:3]

    accz = jnp.where(c == 0, jnp.full((B, _LANES), -1.0, jnp.float32),
                     accz_ref[...])
    acci = jnp.where(c == 0, jnp.zeros((B, _LANES), jnp.int32),
                     acci_ref[...])

    base = c * block_w
    rowc = jax.lax.broadcasted_iota(jnp.uint32, (B, sub_w), 0) \
        * jnp.uint32(n_cols)

    for s in range(block_w // sub_w):
        off = s * sub_w
        xs = x_ref[:, off:off + sub_w]
        a = jnp.abs(xs)
        t = a * c1 + c0
        p = t * t * (_PMAX - _PMIN) + _PMIN
        probs_ref[:, off:off + sub_w] = p * inv_psum

        coli = jax.lax.broadcasted_iota(jnp.int32, (B, sub_w), 1) \
            + (base + off)
        f = rowc + coli.astype(jnp.uint32)
        bits = _threefry_xor(f)
        fb = (bits >> jnp.uint32(9)) | jnp.uint32(0x3F800000)
        fl = jax.lax.bitcast_convert_type(fb, jnp.float32) - jnp.float32(1.0)
        u = jnp.maximum(_TINY, fl * (jnp.float32(1.0) - _TINY) + _TINY)
        w = -jnp.log(u)
        z = p / w

        for k in range(sub_w // _LANES):
            zk = z[:, k * _LANES:(k + 1) * _LANES]
            colk = coli[:, k * _LANES:(k + 1) * _LANES]
            upd = (zk > accz) & (colk < n_cols)
            accz = jnp.where(upd, zk, accz)
            acci = jnp.where(upd, colk, acci)

    accz_ref[...] = accz
    acci_ref[...] = acci

    @pl.when(c == nc - 1)
    def _():
        zmax = jnp.max(accz, axis=-1, keepdims=True)
        li = jnp.min(jnp.where(accz == zmax, acci, jnp.int32(n_cols)),
                     axis=-1, keepdims=True)
        idx_ref[...] = li


def kernel(igrad):
    B, N = igrad.shape
    w1 = 8192
    c1 = pl.cdiv(N, w1)
    stats = pl.pallas_call(
        functools.partial(_stats_kernel, n_cols=N, block_w=w1),
        grid=(c1,),
        in_specs=[pl.BlockSpec((B, w1), lambda c: (0, c))],
        out_specs=pl.BlockSpec((B, 4), lambda c: (0, 0)),
        out_shape=jax.ShapeDtypeStruct((B, 4), jnp.float32),
        scratch_shapes=[pltpu.VMEM((B, _LANES), jnp.float32)
                        for _ in range(4)],
    )(igrad)

    w2 = 4096
    sub_w = 128
    c2 = pl.cdiv(N, w2)
    probs, idx2 = pl.pallas_call(
        functools.partial(_main_kernel, n_cols=N, block_w=w2, sub_w=sub_w),
        grid=(c2,),
        in_specs=[pl.BlockSpec((B, w2), lambda c: (0, c)),
                  pl.BlockSpec((B, 4), lambda c: (0, 0))],
        out_specs=[pl.BlockSpec((B, w2), lambda c: (0, c)),
                   pl.BlockSpec((B, 1), lambda c: (0, 0))],
        out_shape=[jax.ShapeDtypeStruct((B, N), jnp.float32),
                   jax.ShapeDtypeStruct((B, 1), jnp.int32)],
        scratch_shapes=[pltpu.VMEM((B, _LANES), jnp.float32),
                        pltpu.VMEM((B, _LANES), jnp.int32)],
    )(igrad, stats)
    return probs, idx2.reshape(B)


# speedup vs baseline: 1.5394x; 1.0747x over previous
import functools

import jax
import jax.numpy as jnp
import numpy as np
from jax.experimental import pallas as pl
from jax.experimental.pallas import tpu as pltpu

_PMIN = np.float32(0.1)
_PMAX = np.float32(1.0)
_EPS_D = np.float32(1e-12)
_TINY = np.float32(np.finfo(np.float32).tiny)
_LANES = 128


def _stats_kernel(x_ref, out_ref, mn_ref, mx_ref, s1_ref, s2_ref, *,
                  n_cols, block_w):
    c = pl.program_id(0)
    nc = pl.num_programs(0)
    B = x_ref.shape[0]
    nfold = block_w // _LANES

    accmn = jnp.where(c == 0, jnp.full((B, _LANES), jnp.inf, jnp.float32),
                      mn_ref[...])
    accmx = jnp.where(c == 0, jnp.zeros((B, _LANES), jnp.float32),
                      mx_ref[...])
    accs1 = jnp.where(c == 0, jnp.zeros((B, _LANES), jnp.float32),
                      s1_ref[...])
    accs2 = jnp.where(c == 0, jnp.zeros((B, _LANES), jnp.float32),
                      s2_ref[...])

    base = c * block_w
    lane = jax.lax.broadcasted_iota(jnp.int32, (B, _LANES), 1)
    for k in range(nfold):
        a = jnp.abs(x_ref[:, k * _LANES:(k + 1) * _LANES])
        valid = (base + k * _LANES) + lane < n_cols
        am = jnp.where(valid, a, 0.0)
        accmn = jnp.minimum(accmn, jnp.where(valid, a, jnp.inf))
        accmx = jnp.maximum(accmx, am)
        accs1 = accs1 + am
        accs2 = accs2 + am * am

    mn_ref[...] = accmn
    mx_ref[...] = accmx
    s1_ref[...] = accs1
    s2_ref[...] = accs2

    @pl.when(c == nc - 1)
    def _():
        gmin = jnp.min(accmn, axis=-1, keepdims=True)
        gmax = jnp.max(accmx, axis=-1, keepdims=True)
        s1 = jnp.sum(accs1, axis=-1, keepdims=True)
        s2 = jnp.sum(accs2, axis=-1, keepdims=True)
        denom = gmax - gmin + _EPS_D
        inv_denom = 1.0 / denom
        nf = jnp.float32(n_cols)
        sumq = (s2 - 2.0 * gmin * s1 + nf * gmin * gmin) \
            * (inv_denom * inv_denom)
        psum = (_PMAX - _PMIN) * sumq + _PMIN * nf
        out_ref[:, 0:1] = -gmin * inv_denom
        out_ref[:, 1:2] = inv_denom
        out_ref[:, 2:3] = 1.0 / psum
        out_ref[:, 3:4] = psum


def _threefry_xor(x1):
    ks = (jnp.uint32(0), jnp.uint32(1), jnp.uint32(0x1BD11BDB))
    rot = ((13, 15, 26, 6), (17, 29, 16, 24))
    x0 = jnp.zeros_like(x1)
    x1 = x1 + ks[1]
    for i in range(5):
        for r in rot[i % 2]:
            x0 = x0 + x1
            x1 = (x1 << jnp.uint32(r)) | (x1 >> jnp.uint32(32 - r))
            x1 = x1 ^ x0
        x0 = x0 + ks[(i + 1) % 3]
        x1 = x1 + ks[(i + 2) % 3] + jnp.uint32(i + 1)
    return x0 ^ x1


def _main_kernel(x_ref, stats_ref, probs_ref, idx_ref, accz_ref, acci_ref, *,
                 n_cols, block_w, sub_w):
    c = pl.program_id(0)
    nc = pl.num_programs(0)
    B = x_ref.shape[0]

    c0 = stats_ref[:, 0:1]
    c1 = stats_ref[:, 1:2]
    inv_psum = stats_ref[:, 2:3]

    accz = jnp.where(c == 0, jnp.full((B, _LANES), -1.0, jnp.float32),
                     accz_ref[...])
    acci = jnp.where(c == 0, jnp.zeros((B, _LANES), jnp.int32),
                     acci_ref[...])

    base = c * block_w
    rowc = jax.lax.broadcasted_iota(jnp.uint32, (B, sub_w), 0) \
        * jnp.uint32(n_cols)

    for s in range(block_w // sub_w):
        off = s * sub_w
        xs = x_ref[:, off:off + sub_w]
        a = jnp.abs(xs)
        t = a * c1 + c0
        p = t * t * (_PMAX - _PMIN) + _PMIN
        probs_ref[:, off:off + sub_w] = p * inv_psum

        coli = jax.lax.broadcasted_iota(jnp.int32, (B, sub_w), 1) \
            + (base + off)
        f = rowc + coli.astype(jnp.uint32)
        bits = _threefry_xor(f)
        fb = (bits >> jnp.uint32(9)) | jnp.uint32(0x3F800000)
        fl = jax.lax.bitcast_convert_type(fb, jnp.float32) - jnp.float32(1.0)
        u = jnp.maximum(_TINY, fl * (jnp.float32(1.0) - _TINY) + _TINY)
        w = -jnp.log(u)
        z = p / w

        for k in range(sub_w // _LANES):
            zk = z[:, k * _LANES:(k + 1) * _LANES]
            colk = coli[:, k * _LANES:(k + 1) * _LANES]
            upd = (zk > accz) & (colk < n_cols)
            accz = jnp.where(upd, zk, accz)
            acci = jnp.where(upd, colk, acci)

    accz_ref[...] = accz
    acci_ref[...] = acci

    @pl.when(c == nc - 1)
    def _():
        zmax = jnp.max(accz, axis=-1, keepdims=True)
        li = jnp.min(jnp.where(accz == zmax, acci, jnp.int32(n_cols)),
                     axis=-1, keepdims=True)
        idx_ref[...] = li


def kernel(igrad):
    B, N = igrad.shape
    w1 = 32768
    c1 = pl.cdiv(N, w1)
    stats = pl.pallas_call(
        functools.partial(_stats_kernel, n_cols=N, block_w=w1),
        grid=(c1,),
        in_specs=[pl.BlockSpec((B, w1), lambda c: (0, c))],
        out_specs=pl.BlockSpec((B, 4), lambda c: (0, 0)),
        out_shape=jax.ShapeDtypeStruct((B, 4), jnp.float32),
        scratch_shapes=[pltpu.VMEM((B, _LANES), jnp.float32)
                        for _ in range(4)],
    )(igrad)

    w2 = 8192
    sub_w = 128
    c2 = pl.cdiv(N, w2)
    probs, idx2 = pl.pallas_call(
        functools.partial(_main_kernel, n_cols=N, block_w=w2, sub_w=sub_w),
        grid=(c2,),
        in_specs=[pl.BlockSpec((B, w2), lambda c: (0, c)),
                  pl.BlockSpec((B, 4), lambda c: (0, 0))],
        out_specs=[pl.BlockSpec((B, w2), lambda c: (0, c)),
                   pl.BlockSpec((B, 1), lambda c: (0, 0))],
        out_shape=[jax.ShapeDtypeStruct((B, N), jnp.float32),
                   jax.ShapeDtypeStruct((B, 1), jnp.int32)],
        scratch_shapes=[pltpu.VMEM((B, _LANES), jnp.float32),
                        pltpu.VMEM((B, _LANES), jnp.int32)],
    )(igrad, stats)
    return probs, idx2.reshape(B)
